# baseline (speedup 1.0000x reference)
import functools

import jax
import jax.numpy as jnp
from jax import lax
from jax.experimental import pallas as pl
from jax.experimental.pallas import tpu as pltpu
from jax.experimental.pallas import tpu_sc as plsc

N = 10000
D = 128
NH = 8
HID = 16
OUT_CH = 64
REC = D + 2 * NH
NCORES = 2
NSUB = 16
CHUNK = 80

f32 = jnp.float32


def _prep_body(act, w, a_s, a_d, hpack_o, aldp_o, initall_o):
  h = jnp.dot(act[...], w[...], preferred_element_type=f32)
  br = h.shape[0]
  h3 = h.reshape(br, NH, HID)
  als = jnp.sum(h3 * a_s[...][None], axis=-1)
  ald = jnp.sum(h3 * a_d[...][None], axis=-1)
  s = als + ald
  ex0 = jnp.exp(jnp.where(s > 0, s, 0.2 * s))
  hpack_o[...] = jnp.concatenate([h, als, ald], axis=1)
  zald = jnp.zeros_like(ald)
  aldp_o[...] = jnp.concatenate([ald, zald], axis=1)
  selfnum = (h3 * ex0[:, :, None]).reshape(br, D)
  initall_o[0] = jnp.concatenate([selfnum, ex0, zald], axis=1)
  initall_o[1] = jnp.zeros((br, REC), f32)


def _prep(act, w, a_s, a_d):
  br = 2000
  grid = (N // br,)
  return pl.pallas_call(
      _prep_body,
      grid=grid,
      in_specs=[
          pl.BlockSpec((br, D), lambda i: (i, 0)),
          pl.BlockSpec((D, D), lambda i: (0, 0)),
          pl.BlockSpec((NH, HID), lambda i: (0, 0)),
          pl.BlockSpec((NH, HID), lambda i: (0, 0)),
      ],
      out_specs=[
          pl.BlockSpec((br, REC), lambda i: (i, 0)),
          pl.BlockSpec((br, 2 * NH), lambda i: (i, 0)),
          pl.BlockSpec((2, br, REC), lambda i: (0, i, 0)),
      ],
      out_shape=[
          jax.ShapeDtypeStruct((N, REC), f32),
          jax.ShapeDtypeStruct((N, 2 * NH), f32),
          jax.ShapeDtypeStruct((2, N, REC), f32),
      ],
  )(act, w, a_s, a_d)


def _edge_body(nchunks, hpack, aldp, srcs, dsts, initall, acc2,
               acc_sh, srcv, dstv, hsb, aldb, msgb):
  c = lax.axis_index("c")
  s = lax.axis_index("s")
  wid = c * NSUB + s
  rows = N // NSUB
  r0 = s * rows
  pltpu.sync_copy(initall.at[c, pl.ds(r0, rows)], acc_sh.at[pl.ds(r0, rows)])
  plsc.subcore_barrier()

  epw = nchunks * CHUNK
  base_w = wid * epw

  def chunk(i, carry):
    base = base_w + i * CHUNK
    pltpu.sync_copy(srcs.at[pl.ds(base, CHUNK)], srcv)
    pltpu.sync_copy(dsts.at[pl.ds(base, CHUNK)], dstv)
    pltpu.sync_copy(hpack.at[srcv], hsb)
    pltpu.sync_copy(aldp.at[dstv], aldb)

    def exloop(e, carry2):
      srow = hsb[e, pl.ds(D, 16)]
      drow = aldb[e, pl.ds(0, 16)]
      sm = srow + drow
      msgb[e, pl.ds(D, 16)] = jnp.exp(jnp.where(sm > 0, sm, 0.2 * sm))
      return carry2

    lax.fori_loop(0, CHUNK, exloop, 0, unroll=8)

    def msgloop(e, carry2):
      ex = msgb[e, pl.ds(D, 16)]
      for hh in range(NH):
        msgb[e, pl.ds(hh * HID, 16)] = hsb[e, pl.ds(hh * HID, 16)] * ex[hh]
      return carry2

    lax.fori_loop(0, CHUNK, msgloop, 0, unroll=4)

    pltpu.sync_copy(msgb, acc_sh.at[dstv], add=True)
    return carry

  lax.fori_loop(0, nchunks, chunk, 0)
  plsc.subcore_barrier()
  pltpu.sync_copy(acc_sh.at[pl.ds(r0, rows)], acc2.at[c, pl.ds(r0, rows)])


def _edge_pass(hpack, aldp, srcs, dsts, initall):
  e_total = srcs.shape[0]
  epw = e_total // (NCORES * NSUB)
  assert epw * NCORES * NSUB == e_total and epw % CHUNK == 0
  nchunks = epw // CHUNK
  mesh = plsc.VectorSubcoreMesh(core_axis_name="c", subcore_axis_name="s")
  kern = pl.kernel(
      functools.partial(_edge_body, nchunks),
      out_type=jax.ShapeDtypeStruct((2, N, REC), f32),
      mesh=mesh,
      compiler_params=pltpu.CompilerParams(use_tc_tiling_on_sc=False),
      scratch_types=[
          pltpu.VMEM_SHARED((N, REC), f32),
          pltpu.VMEM((CHUNK,), jnp.int32),
          pltpu.VMEM((CHUNK,), jnp.int32),
          pltpu.VMEM((CHUNK, REC), f32),
          pltpu.VMEM((CHUNK, 2 * NH), f32),
          pltpu.VMEM((CHUNK, REC), f32),
      ],
  )
  return kern(hpack, aldp, srcs, dsts, initall)


def _finish_body(acc2, b, o_ref):
  a0 = acc2[0]
  a1 = acc2[1]
  num = a0[:, :D] + a1[:, :D]
  den = a0[:, D:D + NH] + a1[:, D:D + NH]
  br = num.shape[0]
  o3 = num.reshape(br, NH, HID) / (den[:, :, None] + 1e-16)
  o = o3.reshape(br, D) + b[...]
  o_ref[...] = jnp.where(o > 0, o, jnp.exp(o) - 1.0)


def _finish(acc2, b):
  br = 2000
  grid = (N // br,)
  return pl.pallas_call(
      _finish_body,
      grid=grid,
      in_specs=[
          pl.BlockSpec((2, br, REC), lambda i: (0, i, 0)),
          pl.BlockSpec((1, D), lambda i: (0, 0)),
      ],
      out_specs=pl.BlockSpec((br, D), lambda i: (i, 0)),
      out_shape=jax.ShapeDtypeStruct((N, D), f32),
  )(acc2, b)


def _final_body(o0, o1, o2, caw, cab, fcw, fcb, out_ref):
  agg = jnp.zeros_like(o0[...])
  for l, o in enumerate((o0, o1, o2)):
    t = jnp.dot(o[...], caw[...][:, l:l + 1], preferred_element_type=f32)
    t = t + cab[0, l]
    w = 1.0 / (1.0 + jnp.exp(-t))
    agg = agg + o[...] * w
  logits = jnp.dot(agg, fcw[...], preferred_element_type=f32) + fcb[...]
  m = jnp.max(logits, axis=1, keepdims=True)
  z = logits - m
  ls = jnp.log(jnp.sum(jnp.exp(z), axis=1, keepdims=True))
  out_ref[...] = z - ls


def _final(o0, o1, o2, caw, cab, fcw, fcb):
  br = 2000
  grid = (N // br,)
  blk = lambda i: (i, 0)
  rep = lambda i: (0, 0)
  return pl.pallas_call(
      _final_body,
      grid=grid,
      in_specs=[
          pl.BlockSpec((br, D), blk),
          pl.BlockSpec((br, D), blk),
          pl.BlockSpec((br, D), blk),
          pl.BlockSpec((D, 3), rep),
          pl.BlockSpec((1, 3), rep),
          pl.BlockSpec((D, OUT_CH), rep),
          pl.BlockSpec((1, OUT_CH), rep),
      ],
      out_specs=pl.BlockSpec((br, OUT_CH), blk),
      out_shape=jax.ShapeDtypeStruct((N, OUT_CH), f32),
  )(o0, o1, o2, caw, cab, fcw, fcb)


@jax.jit
def kernel(x, edge_index, W0, as0, ad0, b0, caw0, cab0, W1, as1, ad1, b1,
           caw1, cab1, W2, as2, ad2, b2, caw2, cab2, fcw, fcb):
  srcs = edge_index[0]
  dsts = edge_index[1]
  params = [(W0, as0, ad0, b0), (W1, as1, ad1, b1), (W2, as2, ad2, b2)]
  act = x
  outs = []
  for (w, a_s, a_d, b) in params:
    hpack, aldp, initall = _prep(act, w, a_s, a_d)
    acc2 = _edge_pass(hpack, aldp, srcs, dsts, initall)
    act = _finish(acc2, b.reshape(1, D))
    outs.append(act)
  caw = jnp.concatenate([caw0, caw1, caw2], axis=1)
  cab = jnp.stack([cab0, cab1, cab2], axis=1).reshape(1, 3)
  return _final(outs[0], outs[1], outs[2], caw, cab, fcw,
                fcb.reshape(1, OUT_CH))


# speedup vs baseline: 1.8551x; 1.8551x over previous
import functools

import jax
import jax.numpy as jnp
from jax import lax
from jax.experimental import pallas as pl
from jax.experimental.pallas import tpu as pltpu
from jax.experimental.pallas import tpu_sc as plsc

N = 10000
D = 128
NH = 8
HID = 16
OUT_CH = 64
REC = D + 2 * NH
NCORES = 2
NSUB = 16
CHUNK = 80

f32 = jnp.float32


def _prep_body(act, w, a_s, a_d, hpack_o, aldp_o, initall_o):
  h = jnp.dot(act[...], w[...], preferred_element_type=f32)
  br = h.shape[0]
  h3 = h.reshape(br, NH, HID)
  als = jnp.sum(h3 * a_s[...][None], axis=-1)
  ald = jnp.sum(h3 * a_d[...][None], axis=-1)
  s = als + ald
  ex0 = jnp.exp(jnp.where(s > 0, s, 0.2 * s))
  hpack_o[...] = jnp.concatenate([h, als, ald], axis=1)
  zald = jnp.zeros_like(ald)
  aldp_o[...] = jnp.concatenate([ald, zald], axis=1)
  selfnum = (h3 * ex0[:, :, None]).reshape(br, D)
  initall_o[0] = jnp.concatenate([selfnum, ex0, zald], axis=1)
  initall_o[1] = jnp.zeros((br, REC), f32)


def _prep(act, w, a_s, a_d):
  br = 2000
  grid = (N // br,)
  return pl.pallas_call(
      _prep_body,
      grid=grid,
      in_specs=[
          pl.BlockSpec((br, D), lambda i: (i, 0)),
          pl.BlockSpec((D, D), lambda i: (0, 0)),
          pl.BlockSpec((NH, HID), lambda i: (0, 0)),
          pl.BlockSpec((NH, HID), lambda i: (0, 0)),
      ],
      out_specs=[
          pl.BlockSpec((br, REC), lambda i: (i, 0)),
          pl.BlockSpec((br, 2 * NH), lambda i: (i, 0)),
          pl.BlockSpec((2, br, REC), lambda i: (0, i, 0)),
      ],
      out_shape=[
          jax.ShapeDtypeStruct((N, REC), f32),
          jax.ShapeDtypeStruct((N, 2 * NH), f32),
          jax.ShapeDtypeStruct((2, N, REC), f32),
      ],
  )(act, w, a_s, a_d)


def _edge_body(nchunks, hpack, aldp, srcs, dsts, initall, acc2,
               acc_sh, srcv, dstv, hsb, aldb, msgb):
  c = lax.axis_index("c")
  s = lax.axis_index("s")
  wid = c * NSUB + s
  rows = N // NSUB
  r0 = s * rows
  pltpu.sync_copy(initall.at[c, pl.ds(r0, rows)], acc_sh.at[pl.ds(r0, rows)])
  plsc.subcore_barrier()

  epw = nchunks * CHUNK
  base_w = wid * epw

  def chunk(i, carry):
    base = base_w + i * CHUNK
    pltpu.sync_copy(srcs.at[pl.ds(base, CHUNK)], srcv)
    pltpu.sync_copy(dsts.at[pl.ds(base, CHUNK)], dstv)
    pltpu.sync_copy(hpack.at[srcv], hsb)
    pltpu.sync_copy(aldp.at[dstv], aldb)

    @functools.partial(plsc.parallel_loop, 0, CHUNK, unroll=8)
    def exloop(e):
      srow = hsb[e, pl.ds(D, 16)]
      drow = aldb[e, pl.ds(0, 16)]
      sm = srow + drow
      msgb[e, pl.ds(D, 16)] = jnp.exp(jnp.where(sm > 0, sm, 0.2 * sm))

    @functools.partial(plsc.parallel_loop, 0, CHUNK, unroll=4)
    def msgloop(e):
      ex = msgb[e, pl.ds(D, 16)]
      for hh in range(NH):
        msgb[e, pl.ds(hh * HID, 16)] = hsb[e, pl.ds(hh * HID, 16)] * ex[hh]

    pltpu.sync_copy(msgb, acc_sh.at[dstv], add=True)
    return carry

  lax.fori_loop(0, nchunks, chunk, 0)
  plsc.subcore_barrier()
  pltpu.sync_copy(acc_sh.at[pl.ds(r0, rows)], acc2.at[c, pl.ds(r0, rows)])


def _edge_pass(hpack, aldp, srcs, dsts, initall):
  e_total = srcs.shape[0]
  epw = e_total // (NCORES * NSUB)
  assert epw * NCORES * NSUB == e_total and epw % CHUNK == 0
  nchunks = epw // CHUNK
  mesh = plsc.VectorSubcoreMesh(core_axis_name="c", subcore_axis_name="s")
  kern = pl.kernel(
      functools.partial(_edge_body, nchunks),
      out_type=jax.ShapeDtypeStruct((2, N, REC), f32),
      mesh=mesh,
      compiler_params=pltpu.CompilerParams(use_tc_tiling_on_sc=False),
      scratch_types=[
          pltpu.VMEM_SHARED((N, REC), f32),
          pltpu.VMEM((CHUNK,), jnp.int32),
          pltpu.VMEM((CHUNK,), jnp.int32),
          pltpu.VMEM((CHUNK, REC), f32),
          pltpu.VMEM((CHUNK, 2 * NH), f32),
          pltpu.VMEM((CHUNK, REC), f32),
      ],
  )
  return kern(hpack, aldp, srcs, dsts, initall)


def _finish_body(acc2, b, o_ref):
  a0 = acc2[0]
  a1 = acc2[1]
  num = a0[:, :D] + a1[:, :D]
  den = a0[:, D:D + NH] + a1[:, D:D + NH]
  br = num.shape[0]
  o3 = num.reshape(br, NH, HID) / (den[:, :, None] + 1e-16)
  o = o3.reshape(br, D) + b[...]
  o_ref[...] = jnp.where(o > 0, o, jnp.exp(o) - 1.0)


def _finish(acc2, b):
  br = 2000
  grid = (N // br,)
  return pl.pallas_call(
      _finish_body,
      grid=grid,
      in_specs=[
          pl.BlockSpec((2, br, REC), lambda i: (0, i, 0)),
          pl.BlockSpec((1, D), lambda i: (0, 0)),
      ],
      out_specs=pl.BlockSpec((br, D), lambda i: (i, 0)),
      out_shape=jax.ShapeDtypeStruct((N, D), f32),
  )(acc2, b)


def _final_body(o0, o1, o2, caw, cab, fcw, fcb, out_ref):
  agg = jnp.zeros_like(o0[...])
  for l, o in enumerate((o0, o1, o2)):
    t = jnp.dot(o[...], caw[...][:, l:l + 1], preferred_element_type=f32)
    t = t + cab[0, l]
    w = 1.0 / (1.0 + jnp.exp(-t))
    agg = agg + o[...] * w
  logits = jnp.dot(agg, fcw[...], preferred_element_type=f32) + fcb[...]
  m = jnp.max(logits, axis=1, keepdims=True)
  z = logits - m
  ls = jnp.log(jnp.sum(jnp.exp(z), axis=1, keepdims=True))
  out_ref[...] = z - ls


def _final(o0, o1, o2, caw, cab, fcw, fcb):
  br = 2000
  grid = (N // br,)
  blk = lambda i: (i, 0)
  rep = lambda i: (0, 0)
  return pl.pallas_call(
      _final_body,
      grid=grid,
      in_specs=[
          pl.BlockSpec((br, D), blk),
          pl.BlockSpec((br, D), blk),
          pl.BlockSpec((br, D), blk),
          pl.BlockSpec((D, 3), rep),
          pl.BlockSpec((1, 3), rep),
          pl.BlockSpec((D, OUT_CH), rep),
          pl.BlockSpec((1, OUT_CH), rep),
      ],
      out_specs=pl.BlockSpec((br, OUT_CH), blk),
      out_shape=jax.ShapeDtypeStruct((N, OUT_CH), f32),
  )(o0, o1, o2, caw, cab, fcw, fcb)


@jax.jit
def kernel(x, edge_index, W0, as0, ad0, b0, caw0, cab0, W1, as1, ad1, b1,
           caw1, cab1, W2, as2, ad2, b2, caw2, cab2, fcw, fcb):
  srcs = edge_index[0]
  dsts = edge_index[1]
  params = [(W0, as0, ad0, b0), (W1, as1, ad1, b1), (W2, as2, ad2, b2)]
  act = x
  outs = []
  for (w, a_s, a_d, b) in params:
    hpack, aldp, initall = _prep(act, w, a_s, a_d)
    acc2 = _edge_pass(hpack, aldp, srcs, dsts, initall)
    act = _finish(acc2, b.reshape(1, D))
    outs.append(act)
  caw = jnp.concatenate([caw0, caw1, caw2], axis=1)
  cab = jnp.stack([cab0, cab1, cab2], axis=1).reshape(1, 3)
  return _final(outs[0], outs[1], outs[2], caw, cab, fcw,
                fcb.reshape(1, OUT_CH))
